# baseline (speedup 1.0000x reference)
import functools

import jax
import jax.numpy as jnp
from jax import lax
from jax.experimental import pallas as pl
from jax.experimental.pallas import tpu as pltpu
from jax.experimental.pallas import tpu_sc as plsc

_N = 10000
_NPAD = 10112
_F = 128
_E = 320000
_NW = 32
_CH = 128
_NCHUNK = 80
_EPT = _CH * _NCHUNK
_EPAD = _NW * _EPT
_DW = 16
_RPT = _NPAD // 16


_G = 8
_NG = _NCHUNK // _G


def _sc_spread_body(z_hbm, idx_hbm, z128_hbm, parts_hbm,
                    win, buf0, buf1, ws0, ws1, bs0, bs1, acc):
    c = lax.axis_index("c")
    s = lax.axis_index("s")
    wid = s * 2 + c
    pltpu.sync_copy(z128_hbm, acc.at[pl.ds(s * _RPT, _RPT)])
    plsc.subcore_barrier()

    pltpu.async_copy(idx_hbm.at[wid, 0], win.at[0], ws0)
    pltpu.async_copy(idx_hbm.at[wid, 1], win.at[1], ws1)
    pltpu.make_async_copy(idx_hbm.at[wid, 0], win.at[0], ws0).wait()
    pltpu.async_copy(z_hbm.at[win.at[0, 0, 0]], buf0, bs0)

    def gpair(gp, carry):
        for slot in (0, 1):
            g = gp * 2 + slot
            wsem = (ws0, ws1)[slot]
            nsem = (ws0, ws1)[1 - slot]
            for k in range(_G):
                bufA, bsA = ((buf0, bs0), (buf1, bs1))[k % 2]
                bufB, bsB = ((buf0, bs0), (buf1, bs1))[1 - (k % 2)]
                if k == _G - 1:
                    pltpu.make_async_copy(idx_hbm.at[wid, 0],
                                          win.at[1 - slot], nsem).wait()
                pltpu.make_async_copy(z_hbm.at[win.at[slot, 0, k]],
                                      bufA, bsA).wait()
                if k < _G - 1:
                    pltpu.async_copy(z_hbm.at[win.at[slot, 0, k + 1]],
                                     bufB, bsB)
                else:
                    pltpu.async_copy(z_hbm.at[win.at[1 - slot, 0, 0]],
                                     bufB, bsB)
                pltpu.sync_copy(bufA, acc.at[win.at[slot, 1, k]], add=True)
            gnext = jnp.minimum(g + 2, _NG - 1)
            pltpu.async_copy(idx_hbm.at[wid, gnext], win.at[slot], wsem)
        return carry

    lax.fori_loop(0, _NG // 2, gpair, 0)
    pltpu.make_async_copy(z_hbm.at[win.at[0, 0, 0]], buf0, bs0).wait()
    pltpu.make_async_copy(idx_hbm.at[wid, 0], win.at[1], ws1).wait()
    plsc.subcore_barrier()
    pltpu.sync_copy(acc.at[pl.ds(s * _RPT, _RPT)],
                    parts_hbm.at[c, pl.ds(s * _RPT, _RPT)])


@functools.lru_cache(maxsize=None)
def _sc_spread():
    mesh = plsc.VectorSubcoreMesh(core_axis_name="c", subcore_axis_name="s")
    return pl.kernel(
        _sc_spread_body,
        out_type=jax.ShapeDtypeStruct((2, _NPAD, _F), jnp.float32),
        mesh=mesh,
        scratch_types=[pltpu.VMEM((2, 2, _G, _CH), jnp.int32),
                       pltpu.VMEM((_CH, _F), jnp.float32),
                       pltpu.VMEM((_CH, _F), jnp.float32),
                       pltpu.SemaphoreType.DMA,
                       pltpu.SemaphoreType.DMA,
                       pltpu.SemaphoreType.DMA,
                       pltpu.SemaphoreType.DMA,
                       pltpu.VMEM_SHARED((_NPAD, _F), jnp.float32)])


def _sc_degree_body(cid_hbm, ones_hbm, z128_hbm, parts_hbm,
                    cid_v, ones_v, sem, acc):
    c = lax.axis_index("c")
    s = lax.axis_index("s")
    wid = s * 2 + c
    pltpu.sync_copy(cid_hbm.at[wid], cid_v)
    pltpu.sync_copy(ones_hbm, ones_v)
    pltpu.sync_copy(z128_hbm, acc.at[pl.ds(s * _RPT, _RPT)])
    plsc.subcore_barrier()

    def group(g, carry):
        for k in range(8):
            pltpu.async_copy(ones_v, acc.at[cid_v.at[g * 8 + k]], sem,
                             add=True)
        for k in range(8):
            pltpu.make_async_copy(ones_v, acc.at[cid_v.at[g * 8 + k]],
                                  sem).wait()
        return carry

    lax.fori_loop(0, _NCHUNK // 8, group, 0)
    plsc.subcore_barrier()
    pltpu.sync_copy(acc.at[pl.ds(s * _RPT, _RPT)],
                    parts_hbm.at[c, pl.ds(s * _RPT, _RPT)])


@functools.lru_cache(maxsize=None)
def _sc_degree():
    mesh = plsc.VectorSubcoreMesh(core_axis_name="c", subcore_axis_name="s")
    return pl.kernel(
        _sc_degree_body,
        out_type=jax.ShapeDtypeStruct((2, _NPAD, _F), jnp.float32),
        mesh=mesh,
        scratch_types=[pltpu.VMEM((_NCHUNK, _CH), jnp.int32),
                       pltpu.VMEM((_CH, _F), jnp.float32),
                       pltpu.SemaphoreType.DMA,
                       pltpu.VMEM_SHARED((_NPAD, _F), jnp.float32)])


def _tc_row2_body(row_ref, col_ref, row2_ref):
    r, c = row_ref[...], col_ref[...]
    row2_ref[...] = jnp.where(r == c, _N, r)


_tc_row2 = pl.pallas_call(
    _tc_row2_body,
    out_shape=jax.ShapeDtypeStruct((_EPAD // 128, 128), jnp.int32))


def _tc_prep_body(degp_ref, feat_ref, dis_ref, z0_ref):
    deg = degp_ref[0, 0:_N, 0:1] + degp_ref[1, 0:_N, 0:1]
    dis = jnp.where(deg > 0, lax.rsqrt(deg), 0.0)
    dis_ref[0:_N] = dis
    dis_ref[_N:_NPAD] = jnp.zeros((_NPAD - _N, 1), jnp.float32)
    z0_ref[0:_N, :] = dis * feat_ref[...]
    z0_ref[_N:_NPAD, :] = jnp.zeros((_NPAD - _N, _F), jnp.float32)


_tc_prep = pl.pallas_call(
    _tc_prep_body,
    out_shape=[jax.ShapeDtypeStruct((_NPAD, 1), jnp.float32),
               jax.ShapeDtypeStruct((_NPAD, _F), jnp.float32)])


def _tc_scale_body(parts_ref, dis_ref, v_ref):
    d = dis_ref[...]
    v_ref[...] = (d * d) * (parts_ref[0] + parts_ref[1])


_tc_scale = pl.pallas_call(
    _tc_scale_body,
    out_shape=jax.ShapeDtypeStruct((_NPAD, _F), jnp.float32))


def _cheb_combine(x, s1, s2, d, w_ref, b):
    tx1 = -(d * s1)
    tx2 = 2.0 * (d * s2) - x
    return (jnp.dot(x, w_ref[0], preferred_element_type=jnp.float32)
            + jnp.dot(tx1, w_ref[1], preferred_element_type=jnp.float32)
            + jnp.dot(tx2, w_ref[2], preferred_element_type=jnp.float32)
            + b)


def _tc_layer1_body(feat_ref, p1_ref, p2_ref, dis_ref, w_ref, b_ref,
                    g_ref, bb_ref, x1_ref, z1_ref):
    d = dis_ref[0:_N]
    s1 = p1_ref[0, 0:_N, :] + p1_ref[1, 0:_N, :]
    s2 = p2_ref[0, 0:_N, :] + p2_ref[1, 0:_N, :]
    y = _cheb_combine(feat_ref[...], s1, s2, d, w_ref, b_ref[...])
    mean = jnp.mean(y, axis=0, keepdims=True)
    var = jnp.mean((y - mean) ** 2, axis=0, keepdims=True)
    yn = (y - mean) * lax.rsqrt(var + 1e-5) * g_ref[...] + bb_ref[...]
    x1 = yn * (1.0 / (1.0 + jnp.exp(-yn)))
    x1_ref[...] = x1
    z1_ref[0:_N, :] = d * x1
    z1_ref[_N:_NPAD, :] = jnp.zeros((_NPAD - _N, _F), jnp.float32)


_tc_layer1 = pl.pallas_call(
    _tc_layer1_body,
    out_shape=[jax.ShapeDtypeStruct((_N, _F), jnp.float32),
               jax.ShapeDtypeStruct((_NPAD, _F), jnp.float32)])


def _tc_layer2_body(x1_ref, p3_ref, p4_ref, dis_ref, w_ref, b_ref,
                    wf_ref, bf_ref, g_ref, bb_ref, out_ref):
    d = dis_ref[0:_N]
    s3 = p3_ref[0, 0:_N, :] + p3_ref[1, 0:_N, :]
    s4 = p4_ref[0, 0:_N, :] + p4_ref[1, 0:_N, :]
    y = _cheb_combine(x1_ref[...], s3, s4, d, w_ref, b_ref[...])
    x2 = jnp.tanh(y)
    v = jnp.dot(x2, wf_ref[...], preferred_element_type=jnp.float32) + bf_ref[...]
    mu = jnp.mean(v)
    sig2 = jnp.mean((v - mu) ** 2)
    out_ref[...] = (v - mu) * lax.rsqrt(sig2 + 1e-5) * g_ref[...] + bb_ref[...]


_tc_layer2 = pl.pallas_call(
    _tc_layer2_body,
    out_shape=jax.ShapeDtypeStruct((_N, 1), jnp.float32))


def kernel(features, edge_index, W1, b1, bn_g, bn_b, W2, b2, Wf, bf, ln_g, ln_b):
    row = edge_index[0]
    col = edge_index[1]
    pad = _EPAD - _E
    rowp = jnp.concatenate([row, jnp.zeros((pad,), jnp.int32)]).reshape(
        _EPAD // 128, 128)
    colp = jnp.concatenate([col, jnp.zeros((pad,), jnp.int32)]).reshape(
        _EPAD // 128, 128)
    z128 = jnp.zeros((_RPT, _F), jnp.float32)
    ones128 = jnp.ones((_CH, _F), jnp.float32)

    sc_spread = _sc_spread()
    row2 = _tc_row2(rowp, colp).reshape(_NW, _NCHUNK, _CH)
    idxpack = jnp.stack([row2.reshape(_NW, _NG, _G, _CH),
                         colp.reshape(_NW, _NG, _G, _CH)], axis=2)
    degp = _sc_degree()(row2, ones128, z128)
    dis, z0 = _tc_prep(degp, features)

    p1 = sc_spread(z0, idxpack, z128)
    v1 = _tc_scale(p1, dis)
    p2 = sc_spread(v1, idxpack, z128)
    x1, z1 = _tc_layer1(features, p1, p2, dis, W1,
                        b1.reshape(1, -1), bn_g.reshape(1, -1),
                        bn_b.reshape(1, -1))

    p3 = sc_spread(z1, idxpack, z128)
    v3 = _tc_scale(p3, dis)
    p4 = sc_spread(v3, idxpack, z128)
    out = _tc_layer2(x1, p3, p4, dis, W2, b2.reshape(1, -1),
                     Wf, bf.reshape(1, 1),
                     ln_g.reshape(-1, 1), ln_b.reshape(-1, 1))
    return out.reshape(-1)


# speedup vs baseline: 1.0098x; 1.0098x over previous
import functools

import jax
import jax.numpy as jnp
from jax import lax
from jax.experimental import pallas as pl
from jax.experimental.pallas import tpu as pltpu
from jax.experimental.pallas import tpu_sc as plsc

_N = 10000
_NPAD = 10112
_F = 128
_E = 320000
_NW = 32
_CH = 128
_NCHUNK = 80
_EPT = _CH * _NCHUNK
_EPAD = _NW * _EPT
_DW = 16
_RPT = _NPAD // 16


def _sc_spread_body(z_hbm, rid_hbm, cid_hbm, z128_hbm, parts_hbm,
                    rid_v, cid_v, buf, acc):
    c = lax.axis_index("c")
    s = lax.axis_index("s")
    wid = s * 2 + c
    pltpu.sync_copy(rid_hbm.at[wid], rid_v)
    pltpu.sync_copy(cid_hbm.at[wid], cid_v)
    pltpu.sync_copy(z128_hbm, acc.at[pl.ds(s * _RPT, _RPT)])
    plsc.subcore_barrier()

    def chunk(j, carry):
        pltpu.sync_copy(z_hbm.at[rid_v.at[j]], buf)
        pltpu.sync_copy(buf, acc.at[cid_v.at[j]], add=True)
        return carry

    lax.fori_loop(0, _NCHUNK, chunk, 0)
    plsc.subcore_barrier()
    pltpu.sync_copy(acc.at[pl.ds(s * _RPT, _RPT)],
                    parts_hbm.at[c, pl.ds(s * _RPT, _RPT)])


@functools.lru_cache(maxsize=None)
def _sc_spread():
    mesh = plsc.VectorSubcoreMesh(core_axis_name="c", subcore_axis_name="s")
    return pl.kernel(
        _sc_spread_body,
        out_type=jax.ShapeDtypeStruct((2, _NPAD, _F), jnp.float32),
        mesh=mesh,
        scratch_types=[pltpu.VMEM((_NCHUNK, _CH), jnp.int32),
                       pltpu.VMEM((_NCHUNK, _CH), jnp.int32),
                       pltpu.VMEM((_CH, _F), jnp.float32),
                       pltpu.VMEM_SHARED((_NPAD, _F), jnp.float32)])


def _sc_degree_body(cid_hbm, ones_hbm, z128_hbm, parts_hbm,
                    cid_v, ones_v, sem, acc):
    c = lax.axis_index("c")
    s = lax.axis_index("s")
    wid = s * 2 + c
    pltpu.sync_copy(cid_hbm.at[wid], cid_v)
    pltpu.sync_copy(ones_hbm, ones_v)
    pltpu.sync_copy(z128_hbm, acc.at[pl.ds(s * _RPT, _RPT)])
    plsc.subcore_barrier()

    def group(g, carry):
        for k in range(8):
            pltpu.async_copy(ones_v, acc.at[cid_v.at[g * 8 + k]], sem,
                             add=True)
        for k in range(8):
            pltpu.make_async_copy(ones_v, acc.at[cid_v.at[g * 8 + k]],
                                  sem).wait()
        return carry

    lax.fori_loop(0, _NCHUNK // 8, group, 0)
    plsc.subcore_barrier()
    pltpu.sync_copy(acc.at[pl.ds(s * _RPT, _RPT)],
                    parts_hbm.at[c, pl.ds(s * _RPT, _RPT)])


@functools.lru_cache(maxsize=None)
def _sc_degree():
    mesh = plsc.VectorSubcoreMesh(core_axis_name="c", subcore_axis_name="s")
    return pl.kernel(
        _sc_degree_body,
        out_type=jax.ShapeDtypeStruct((2, _NPAD, _F), jnp.float32),
        mesh=mesh,
        scratch_types=[pltpu.VMEM((_NCHUNK, _CH), jnp.int32),
                       pltpu.VMEM((_CH, _F), jnp.float32),
                       pltpu.SemaphoreType.DMA,
                       pltpu.VMEM_SHARED((_NPAD, _F), jnp.float32)])


def _tc_row2_body(row_ref, col_ref, row2_ref):
    r, c = row_ref[...], col_ref[...]
    row2_ref[...] = jnp.where(r == c, _N, r)


_tc_row2 = pl.pallas_call(
    _tc_row2_body,
    out_shape=jax.ShapeDtypeStruct((_EPAD // 128, 128), jnp.int32))


def _tc_prep_body(degp_ref, feat_ref, dis_ref, z0_ref):
    deg = degp_ref[0, 0:_N, 0:1] + degp_ref[1, 0:_N, 0:1]
    dis = jnp.where(deg > 0, lax.rsqrt(deg), 0.0)
    dis_ref[0:_N] = dis
    dis_ref[_N:_NPAD] = jnp.zeros((_NPAD - _N, 1), jnp.float32)
    z0_ref[0:_N, :] = dis * feat_ref[...]
    z0_ref[_N:_NPAD, :] = jnp.zeros((_NPAD - _N, _F), jnp.float32)


_tc_prep = pl.pallas_call(
    _tc_prep_body,
    out_shape=[jax.ShapeDtypeStruct((_NPAD, 1), jnp.float32),
               jax.ShapeDtypeStruct((_NPAD, _F), jnp.float32)])


def _tc_scale_body(parts_ref, dis_ref, v_ref):
    d = dis_ref[...]
    v_ref[...] = (d * d) * (parts_ref[0] + parts_ref[1])


_tc_scale = pl.pallas_call(
    _tc_scale_body,
    out_shape=jax.ShapeDtypeStruct((_NPAD, _F), jnp.float32))


def _cheb_combine(x, s1, s2, d, w_ref, b):
    tx1 = -(d * s1)
    tx2 = 2.0 * (d * s2) - x
    return (jnp.dot(x, w_ref[0], preferred_element_type=jnp.float32)
            + jnp.dot(tx1, w_ref[1], preferred_element_type=jnp.float32)
            + jnp.dot(tx2, w_ref[2], preferred_element_type=jnp.float32)
            + b)


def _tc_layer1_body(feat_ref, p1_ref, p2_ref, dis_ref, w_ref, b_ref,
                    g_ref, bb_ref, x1_ref, z1_ref):
    d = dis_ref[0:_N]
    s1 = p1_ref[0, 0:_N, :] + p1_ref[1, 0:_N, :]
    s2 = p2_ref[0, 0:_N, :] + p2_ref[1, 0:_N, :]
    y = _cheb_combine(feat_ref[...], s1, s2, d, w_ref, b_ref[...])
    mean = jnp.mean(y, axis=0, keepdims=True)
    var = jnp.mean((y - mean) ** 2, axis=0, keepdims=True)
    yn = (y - mean) * lax.rsqrt(var + 1e-5) * g_ref[...] + bb_ref[...]
    x1 = yn * (1.0 / (1.0 + jnp.exp(-yn)))
    x1_ref[...] = x1
    z1_ref[0:_N, :] = d * x1
    z1_ref[_N:_NPAD, :] = jnp.zeros((_NPAD - _N, _F), jnp.float32)


_tc_layer1 = pl.pallas_call(
    _tc_layer1_body,
    out_shape=[jax.ShapeDtypeStruct((_N, _F), jnp.float32),
               jax.ShapeDtypeStruct((_NPAD, _F), jnp.float32)])


def _tc_layer2_body(x1_ref, p3_ref, p4_ref, dis_ref, w_ref, b_ref,
                    wf_ref, bf_ref, g_ref, bb_ref, out_ref):
    d = dis_ref[0:_N]
    s3 = p3_ref[0, 0:_N, :] + p3_ref[1, 0:_N, :]
    s4 = p4_ref[0, 0:_N, :] + p4_ref[1, 0:_N, :]
    y = _cheb_combine(x1_ref[...], s3, s4, d, w_ref, b_ref[...])
    x2 = jnp.tanh(y)
    v = jnp.dot(x2, wf_ref[...], preferred_element_type=jnp.float32) + bf_ref[...]
    mu = jnp.mean(v)
    sig2 = jnp.mean((v - mu) ** 2)
    out_ref[...] = (v - mu) * lax.rsqrt(sig2 + 1e-5) * g_ref[...] + bb_ref[...]


_tc_layer2 = pl.pallas_call(
    _tc_layer2_body,
    out_shape=jax.ShapeDtypeStruct((_N, 1), jnp.float32))


def kernel(features, edge_index, W1, b1, bn_g, bn_b, W2, b2, Wf, bf, ln_g, ln_b):
    row = edge_index[0]
    col = edge_index[1]
    pad = _EPAD - _E
    rowp = jnp.concatenate([row, jnp.zeros((pad,), jnp.int32)]).reshape(
        _EPAD // 128, 128)
    colp = jnp.concatenate([col, jnp.zeros((pad,), jnp.int32)]).reshape(
        _EPAD // 128, 128)
    z128 = jnp.zeros((_RPT, _F), jnp.float32)
    ones128 = jnp.ones((_CH, _F), jnp.float32)

    sc_spread = _sc_spread()
    row2 = _tc_row2(rowp, colp).reshape(_NW, _NCHUNK, _CH)
    colp = colp.reshape(_NW, _NCHUNK, _CH)
    degp = _sc_degree()(row2, ones128, z128)
    dis, z0 = _tc_prep(degp, features)

    p1 = sc_spread(z0, row2, colp, z128)
    v1 = _tc_scale(p1, dis)
    p2 = sc_spread(v1, row2, colp, z128)
    x1, z1 = _tc_layer1(features, p1, p2, dis, W1,
                        b1.reshape(1, -1), bn_g.reshape(1, -1),
                        bn_b.reshape(1, -1))

    p3 = sc_spread(z1, row2, colp, z128)
    v3 = _tc_scale(p3, dis)
    p4 = sc_spread(v3, row2, colp, z128)
    out = _tc_layer2(x1, p3, p4, dis, W2, b2.reshape(1, -1),
                     Wf, bf.reshape(1, 1),
                     ln_g.reshape(-1, 1), ln_b.reshape(-1, 1))
    return out.reshape(-1)


# speedup vs baseline: 2.5539x; 2.5291x over previous
import functools

import jax
import jax.numpy as jnp
from jax import lax
from jax.experimental import pallas as pl
from jax.experimental.pallas import tpu as pltpu
from jax.experimental.pallas import tpu_sc as plsc

_N = 10000
_NPAD = 10112
_F = 128
_E = 320000
_NW = 32
_CH = 128
_NCHUNK = 80
_EPT = _CH * _NCHUNK
_EPAD = _NW * _EPT
_DW = 16
_RPT = _NPAD // 16


def _sc_spread_body(z_hbm, rid_hbm, cid_hbm, z128_hbm, parts_hbm,
                    rid_v, cid_v, buf, acc):
    c = lax.axis_index("c")
    s = lax.axis_index("s")
    wid = s * 2 + c
    pltpu.sync_copy(rid_hbm.at[wid], rid_v)
    pltpu.sync_copy(cid_hbm.at[wid], cid_v)
    pltpu.sync_copy(z128_hbm, acc.at[pl.ds(s * _RPT, _RPT)])
    plsc.subcore_barrier()

    def chunk(j, carry):
        pltpu.sync_copy(z_hbm.at[rid_v.at[j]], buf)
        pltpu.sync_copy(buf, acc.at[cid_v.at[j]], add=True)
        return carry

    lax.fori_loop(0, _NCHUNK, chunk, 0)
    plsc.subcore_barrier()
    pltpu.sync_copy(acc.at[pl.ds(s * _RPT, _RPT)],
                    parts_hbm.at[c, pl.ds(s * _RPT, _RPT)])


@functools.lru_cache(maxsize=None)
def _sc_spread():
    mesh = plsc.VectorSubcoreMesh(core_axis_name="c", subcore_axis_name="s")
    return pl.kernel(
        _sc_spread_body,
        out_type=jax.ShapeDtypeStruct((2, _NPAD, _F), jnp.float32),
        mesh=mesh,
        scratch_types=[pltpu.VMEM((_NCHUNK, _CH), jnp.int32),
                       pltpu.VMEM((_NCHUNK, _CH), jnp.int32),
                       pltpu.VMEM((_CH, _F), jnp.float32),
                       pltpu.VMEM_SHARED((_NPAD, _F), jnp.float32)])


def _sc_degree_body(cid_hbm, ones_hbm, z128_hbm, parts_hbm,
                    cid_v, ones_v, sem, acc):
    c = lax.axis_index("c")
    s = lax.axis_index("s")
    wid = s * 2 + c
    pltpu.sync_copy(cid_hbm.at[wid], cid_v)
    pltpu.sync_copy(ones_hbm, ones_v)
    pltpu.sync_copy(z128_hbm, acc.at[pl.ds(s * _RPT, _RPT)])
    plsc.subcore_barrier()

    def group(g, carry):
        for k in range(8):
            pltpu.async_copy(ones_v, acc.at[cid_v.at[g * 8 + k]], sem,
                             add=True)
        for k in range(8):
            pltpu.make_async_copy(ones_v, acc.at[cid_v.at[g * 8 + k]],
                                  sem).wait()
        return carry

    lax.fori_loop(0, _NCHUNK // 8, group, 0)
    plsc.subcore_barrier()
    pltpu.sync_copy(acc.at[pl.ds(s * _RPT, _RPT)],
                    parts_hbm.at[c, pl.ds(s * _RPT, _RPT)])


@functools.lru_cache(maxsize=None)
def _sc_degree():
    mesh = plsc.VectorSubcoreMesh(core_axis_name="c", subcore_axis_name="s")
    return pl.kernel(
        _sc_degree_body,
        out_type=jax.ShapeDtypeStruct((2, _NPAD, _F), jnp.float32),
        mesh=mesh,
        scratch_types=[pltpu.VMEM((_NCHUNK, _CH), jnp.int32),
                       pltpu.VMEM((_CH, _F), jnp.float32),
                       pltpu.SemaphoreType.DMA,
                       pltpu.VMEM_SHARED((_NPAD, _F), jnp.float32)])


def _tc_row2_body(row_ref, col_ref, row2_ref, cid2_ref):
    r, c = row_ref[...], col_ref[...]
    row2_ref[...] = jnp.where(r == c, _N, r)
    cid2_ref[...] = jnp.where(r == c, _N, c)


_tc_row2 = pl.pallas_call(
    _tc_row2_body,
    out_shape=[jax.ShapeDtypeStruct((_EPAD // 128, 128), jnp.int32),
               jax.ShapeDtypeStruct((_EPAD // 128, 128), jnp.int32)])


def _tc_prep_body(degp_ref, feat_ref, dis_ref, z0_ref):
    deg = degp_ref[0, 0:_N, 0:1] + degp_ref[1, 0:_N, 0:1]
    dis = jnp.where(deg > 0, lax.rsqrt(deg), 0.0)
    dis_ref[0:_N] = dis
    dis_ref[_N:_NPAD] = jnp.zeros((_NPAD - _N, 1), jnp.float32)
    z0_ref[0:_N, :] = dis * feat_ref[...]
    z0_ref[_N:_NPAD, :] = jnp.zeros((_NPAD - _N, _F), jnp.float32)


_tc_prep = pl.pallas_call(
    _tc_prep_body,
    out_shape=[jax.ShapeDtypeStruct((_NPAD, 1), jnp.float32),
               jax.ShapeDtypeStruct((_NPAD, _F), jnp.float32)])


def _tc_scale_body(parts_ref, dis_ref, v_ref):
    d = dis_ref[...]
    v_ref[...] = (d * d) * (parts_ref[0] + parts_ref[1])


_tc_scale = pl.pallas_call(
    _tc_scale_body,
    out_shape=jax.ShapeDtypeStruct((_NPAD, _F), jnp.float32))


def _cheb_combine(x, s1, s2, d, w_ref, b):
    tx1 = -(d * s1)
    tx2 = 2.0 * (d * s2) - x
    return (jnp.dot(x, w_ref[0], preferred_element_type=jnp.float32)
            + jnp.dot(tx1, w_ref[1], preferred_element_type=jnp.float32)
            + jnp.dot(tx2, w_ref[2], preferred_element_type=jnp.float32)
            + b)


def _tc_layer1_body(feat_ref, p1_ref, p2_ref, dis_ref, w_ref, b_ref,
                    g_ref, bb_ref, x1_ref, z1_ref):
    d = dis_ref[0:_N]
    s1 = p1_ref[0, 0:_N, :] + p1_ref[1, 0:_N, :]
    s2 = p2_ref[0, 0:_N, :] + p2_ref[1, 0:_N, :]
    y = _cheb_combine(feat_ref[...], s1, s2, d, w_ref, b_ref[...])
    mean = jnp.mean(y, axis=0, keepdims=True)
    var = jnp.mean((y - mean) ** 2, axis=0, keepdims=True)
    yn = (y - mean) * lax.rsqrt(var + 1e-5) * g_ref[...] + bb_ref[...]
    x1 = yn * (1.0 / (1.0 + jnp.exp(-yn)))
    x1_ref[...] = x1
    z1_ref[0:_N, :] = d * x1
    z1_ref[_N:_NPAD, :] = jnp.zeros((_NPAD - _N, _F), jnp.float32)


_tc_layer1 = pl.pallas_call(
    _tc_layer1_body,
    out_shape=[jax.ShapeDtypeStruct((_N, _F), jnp.float32),
               jax.ShapeDtypeStruct((_NPAD, _F), jnp.float32)])


def _tc_layer2_body(x1_ref, p3_ref, p4_ref, dis_ref, w_ref, b_ref,
                    wf_ref, bf_ref, g_ref, bb_ref, out_ref):
    d = dis_ref[0:_N]
    s3 = p3_ref[0, 0:_N, :] + p3_ref[1, 0:_N, :]
    s4 = p4_ref[0, 0:_N, :] + p4_ref[1, 0:_N, :]
    y = _cheb_combine(x1_ref[...], s3, s4, d, w_ref, b_ref[...])
    x2 = jnp.tanh(y)
    v = jnp.dot(x2, wf_ref[...], preferred_element_type=jnp.float32) + bf_ref[...]
    mu = jnp.mean(v)
    sig2 = jnp.mean((v - mu) ** 2)
    out_ref[...] = (v - mu) * lax.rsqrt(sig2 + 1e-5) * g_ref[...] + bb_ref[...]


_tc_layer2 = pl.pallas_call(
    _tc_layer2_body,
    out_shape=jax.ShapeDtypeStruct((_N, 1), jnp.float32))


def kernel(features, edge_index, W1, b1, bn_g, bn_b, W2, b2, Wf, bf, ln_g, ln_b):
    row = edge_index[0]
    col = edge_index[1]
    pad = _EPAD - _E
    padidx = jnp.arange(pad, dtype=jnp.int32) % _N
    rowp = jnp.concatenate([row, padidx]).reshape(_EPAD // 128, 128)
    colp = jnp.concatenate([col, padidx]).reshape(_EPAD // 128, 128)
    z128 = jnp.zeros((_RPT, _F), jnp.float32)
    ones128 = jnp.ones((_CH, _F), jnp.float32)

    sc_spread = _sc_spread()
    row2, cid2 = _tc_row2(rowp, colp)
    row2 = row2.reshape(_NW, _NCHUNK, _CH)
    cid2 = cid2.reshape(_NW, _NCHUNK, _CH)
    ridp = rowp.reshape(_NW, _NCHUNK, _CH)
    degp = _sc_degree()(row2, ones128, z128)
    dis, z0 = _tc_prep(degp, features)

    p1 = sc_spread(z0, ridp, cid2, z128)
    v1 = _tc_scale(p1, dis)
    p2 = sc_spread(v1, ridp, cid2, z128)
    x1, z1 = _tc_layer1(features, p1, p2, dis, W1,
                        b1.reshape(1, -1), bn_g.reshape(1, -1),
                        bn_b.reshape(1, -1))

    p3 = sc_spread(z1, ridp, cid2, z128)
    v3 = _tc_scale(p3, dis)
    p4 = sc_spread(v3, ridp, cid2, z128)
    out = _tc_layer2(x1, p3, p4, dis, W2, b2.reshape(1, -1),
                     Wf, bf.reshape(1, 1),
                     ln_g.reshape(-1, 1), ln_b.reshape(-1, 1))
    return out.reshape(-1)


# speedup vs baseline: 3.2170x; 1.2596x over previous
import functools

import jax
import jax.numpy as jnp
from jax import lax
from jax.experimental import pallas as pl
from jax.experimental.pallas import tpu as pltpu
from jax.experimental.pallas import tpu_sc as plsc

_N = 10000
_NPAD = 10112
_F = 128
_E = 320000
_NW = 32
_CH = 128
_NCHUNK = 80
_EPT = _CH * _NCHUNK
_EPAD = _NW * _EPT
_DW = 16
_RPT = _NPAD // 16


_G = 8
_NG = _NCHUNK // _G


def _sc_spread_body(z_hbm, idx_hbm, z128_hbm, parts_hbm,
                    win, buf0, buf1, ws0, ws1, bs0, bs1, acc):
    c = lax.axis_index("c")
    s = lax.axis_index("s")
    wid = s * 2 + c
    pltpu.sync_copy(z128_hbm, acc.at[pl.ds(s * _RPT, _RPT)])
    plsc.subcore_barrier()

    pltpu.async_copy(idx_hbm.at[wid, 0], win.at[0], ws0)
    pltpu.async_copy(idx_hbm.at[wid, 1], win.at[1], ws1)
    pltpu.make_async_copy(idx_hbm.at[wid, 0], win.at[0], ws0).wait()
    pltpu.async_copy(z_hbm.at[win.at[0, 0, 0]], buf0, bs0)

    def gpair(gp, carry):
        for slot in (0, 1):
            g = gp * 2 + slot
            wsem = (ws0, ws1)[slot]
            nsem = (ws0, ws1)[1 - slot]
            for k in range(_G):
                bufA, bsA = ((buf0, bs0), (buf1, bs1))[k % 2]
                bufB, bsB = ((buf0, bs0), (buf1, bs1))[1 - (k % 2)]
                if k == _G - 1:
                    pltpu.make_async_copy(idx_hbm.at[wid, 0],
                                          win.at[1 - slot], nsem).wait()
                pltpu.make_async_copy(z_hbm.at[win.at[slot, 0, k]],
                                      bufA, bsA).wait()
                if k < _G - 1:
                    pltpu.async_copy(z_hbm.at[win.at[slot, 0, k + 1]],
                                     bufB, bsB)
                else:
                    pltpu.async_copy(z_hbm.at[win.at[1 - slot, 0, 0]],
                                     bufB, bsB)
                pltpu.sync_copy(bufA, acc.at[win.at[slot, 1, k]], add=True)
            gnext = jnp.minimum(g + 2, _NG - 1)
            pltpu.async_copy(idx_hbm.at[wid, gnext], win.at[slot], wsem)
        return carry

    lax.fori_loop(0, _NG // 2, gpair, 0)
    pltpu.make_async_copy(z_hbm.at[win.at[0, 0, 0]], buf0, bs0).wait()
    pltpu.make_async_copy(idx_hbm.at[wid, 0], win.at[1], ws1).wait()
    plsc.subcore_barrier()
    pltpu.sync_copy(acc.at[pl.ds(s * _RPT, _RPT)],
                    parts_hbm.at[c, pl.ds(s * _RPT, _RPT)])


@functools.lru_cache(maxsize=None)
def _sc_spread():
    mesh = plsc.VectorSubcoreMesh(core_axis_name="c", subcore_axis_name="s")
    return pl.kernel(
        _sc_spread_body,
        out_type=jax.ShapeDtypeStruct((2, _NPAD, _F), jnp.float32),
        mesh=mesh,
        scratch_types=[pltpu.VMEM((2, 2, _G, _CH), jnp.int32),
                       pltpu.VMEM((_CH, _F), jnp.float32),
                       pltpu.VMEM((_CH, _F), jnp.float32),
                       pltpu.SemaphoreType.DMA,
                       pltpu.SemaphoreType.DMA,
                       pltpu.SemaphoreType.DMA,
                       pltpu.SemaphoreType.DMA,
                       pltpu.VMEM_SHARED((_NPAD, _F), jnp.float32)])


def _sc_degree_body(cid_hbm, ones_hbm, z128_hbm, parts_hbm,
                    cid_v, ones_v, sem, acc):
    c = lax.axis_index("c")
    s = lax.axis_index("s")
    wid = s * 2 + c
    pltpu.sync_copy(cid_hbm.at[wid], cid_v)
    pltpu.sync_copy(ones_hbm, ones_v)
    pltpu.sync_copy(z128_hbm, acc.at[pl.ds(s * _RPT, _RPT)])
    plsc.subcore_barrier()

    def group(g, carry):
        for k in range(8):
            pltpu.async_copy(ones_v, acc.at[cid_v.at[g * 8 + k]], sem,
                             add=True)
        for k in range(8):
            pltpu.make_async_copy(ones_v, acc.at[cid_v.at[g * 8 + k]],
                                  sem).wait()
        return carry

    lax.fori_loop(0, _NCHUNK // 8, group, 0)
    plsc.subcore_barrier()
    pltpu.sync_copy(acc.at[pl.ds(s * _RPT, _RPT)],
                    parts_hbm.at[c, pl.ds(s * _RPT, _RPT)])


@functools.lru_cache(maxsize=None)
def _sc_degree():
    mesh = plsc.VectorSubcoreMesh(core_axis_name="c", subcore_axis_name="s")
    return pl.kernel(
        _sc_degree_body,
        out_type=jax.ShapeDtypeStruct((2, _NPAD, _F), jnp.float32),
        mesh=mesh,
        scratch_types=[pltpu.VMEM((_NCHUNK, _CH), jnp.int32),
                       pltpu.VMEM((_CH, _F), jnp.float32),
                       pltpu.SemaphoreType.DMA,
                       pltpu.VMEM_SHARED((_NPAD, _F), jnp.float32)])


def _tc_row2_body(row_ref, col_ref, row2_ref, cid2_ref):
    r, c = row_ref[...], col_ref[...]
    row2_ref[...] = jnp.where(r == c, _N, r)
    cid2_ref[...] = jnp.where(r == c, _N, c)


_tc_row2 = pl.pallas_call(
    _tc_row2_body,
    out_shape=[jax.ShapeDtypeStruct((_EPAD // 128, 128), jnp.int32),
               jax.ShapeDtypeStruct((_EPAD // 128, 128), jnp.int32)])


def _tc_prep_body(degp_ref, feat_ref, dis_ref, z0_ref):
    deg = degp_ref[0, 0:_N, 0:1] + degp_ref[1, 0:_N, 0:1]
    dis = jnp.where(deg > 0, lax.rsqrt(deg), 0.0)
    dis_ref[0:_N] = dis
    dis_ref[_N:_NPAD] = jnp.zeros((_NPAD - _N, 1), jnp.float32)
    z0_ref[0:_N, :] = dis * feat_ref[...]
    z0_ref[_N:_NPAD, :] = jnp.zeros((_NPAD - _N, _F), jnp.float32)


_tc_prep = pl.pallas_call(
    _tc_prep_body,
    out_shape=[jax.ShapeDtypeStruct((_NPAD, 1), jnp.float32),
               jax.ShapeDtypeStruct((_NPAD, _F), jnp.float32)])


def _tc_scale_body(parts_ref, dis_ref, v_ref):
    d = dis_ref[...]
    v_ref[...] = (d * d) * (parts_ref[0] + parts_ref[1])


_tc_scale = pl.pallas_call(
    _tc_scale_body,
    out_shape=jax.ShapeDtypeStruct((_NPAD, _F), jnp.float32))


def _cheb_combine(x, s1, s2, d, w_ref, b):
    tx1 = -(d * s1)
    tx2 = 2.0 * (d * s2) - x
    return (jnp.dot(x, w_ref[0], preferred_element_type=jnp.float32)
            + jnp.dot(tx1, w_ref[1], preferred_element_type=jnp.float32)
            + jnp.dot(tx2, w_ref[2], preferred_element_type=jnp.float32)
            + b)


def _tc_layer1_body(feat_ref, p1_ref, p2_ref, dis_ref, w_ref, b_ref,
                    g_ref, bb_ref, x1_ref, z1_ref):
    d = dis_ref[0:_N]
    s1 = p1_ref[0, 0:_N, :] + p1_ref[1, 0:_N, :]
    s2 = p2_ref[0, 0:_N, :] + p2_ref[1, 0:_N, :]
    y = _cheb_combine(feat_ref[...], s1, s2, d, w_ref, b_ref[...])
    mean = jnp.mean(y, axis=0, keepdims=True)
    var = jnp.mean((y - mean) ** 2, axis=0, keepdims=True)
    yn = (y - mean) * lax.rsqrt(var + 1e-5) * g_ref[...] + bb_ref[...]
    x1 = yn * (1.0 / (1.0 + jnp.exp(-yn)))
    x1_ref[...] = x1
    z1_ref[0:_N, :] = d * x1
    z1_ref[_N:_NPAD, :] = jnp.zeros((_NPAD - _N, _F), jnp.float32)


_tc_layer1 = pl.pallas_call(
    _tc_layer1_body,
    out_shape=[jax.ShapeDtypeStruct((_N, _F), jnp.float32),
               jax.ShapeDtypeStruct((_NPAD, _F), jnp.float32)])


def _tc_layer2_body(x1_ref, p3_ref, p4_ref, dis_ref, w_ref, b_ref,
                    wf_ref, bf_ref, g_ref, bb_ref, out_ref):
    d = dis_ref[0:_N]
    s3 = p3_ref[0, 0:_N, :] + p3_ref[1, 0:_N, :]
    s4 = p4_ref[0, 0:_N, :] + p4_ref[1, 0:_N, :]
    y = _cheb_combine(x1_ref[...], s3, s4, d, w_ref, b_ref[...])
    x2 = jnp.tanh(y)
    v = jnp.dot(x2, wf_ref[...], preferred_element_type=jnp.float32) + bf_ref[...]
    mu = jnp.mean(v)
    sig2 = jnp.mean((v - mu) ** 2)
    out_ref[...] = (v - mu) * lax.rsqrt(sig2 + 1e-5) * g_ref[...] + bb_ref[...]


_tc_layer2 = pl.pallas_call(
    _tc_layer2_body,
    out_shape=jax.ShapeDtypeStruct((_N, 1), jnp.float32))


def kernel(features, edge_index, W1, b1, bn_g, bn_b, W2, b2, Wf, bf, ln_g, ln_b):
    row = edge_index[0]
    col = edge_index[1]
    pad = _EPAD - _E
    padidx = jnp.arange(pad, dtype=jnp.int32) % _N
    rowp = jnp.concatenate([row, padidx]).reshape(_EPAD // 128, 128)
    colp = jnp.concatenate([col, padidx]).reshape(_EPAD // 128, 128)
    z128 = jnp.zeros((_RPT, _F), jnp.float32)
    ones128 = jnp.ones((_CH, _F), jnp.float32)

    sc_spread = _sc_spread()
    row2, cid2 = _tc_row2(rowp, colp)
    row2 = row2.reshape(_NW, _NCHUNK, _CH)
    idxpack = jnp.stack([rowp.reshape(_NW, _NG, _G, _CH),
                         cid2.reshape(_NW, _NG, _G, _CH)], axis=2)
    degp = _sc_degree()(row2, ones128, z128)
    dis, z0 = _tc_prep(degp, features)

    p1 = sc_spread(z0, idxpack, z128)
    v1 = _tc_scale(p1, dis)
    p2 = sc_spread(v1, idxpack, z128)
    x1, z1 = _tc_layer1(features, p1, p2, dis, W1,
                        b1.reshape(1, -1), bn_g.reshape(1, -1),
                        bn_b.reshape(1, -1))

    p3 = sc_spread(z1, idxpack, z128)
    v3 = _tc_scale(p3, dis)
    p4 = sc_spread(v3, idxpack, z128)
    out = _tc_layer2(x1, p3, p4, dis, W2, b2.reshape(1, -1),
                     Wf, bf.reshape(1, 1),
                     ln_g.reshape(-1, 1), ln_b.reshape(-1, 1))
    return out.reshape(-1)
